# baseline (speedup 1.0000x reference)
import jax
import jax.numpy as jnp
from jax import lax
from jax.experimental import pallas as pl
from jax.experimental.pallas import tpu as pltpu
from jax.experimental.pallas import tpu_sc as plsc

VOCAB = 128
D = 128
B = 16384
NC = 2
NS = 16
NW = NC * NS
BPW = B // NW
CHUNK = 128
NCHUNK = BPW // CHUNK


def _body(pos_hbm, row_hbm, col_hbm, out_hbm,
          pos_v, ctab_v, ridx_v, cidx_v, out_v,
          sem_pos, sem_tab, sem_g, sem_out):
    wid = lax.axis_index("s") * NC + lax.axis_index("c")
    base = wid * BPW

    pos_cps = [
        pltpu.async_copy(pos_hbm.at[a, pl.ds(base, BPW)], pos_v.at[a],
                         sem_pos)
        for a in range(4)]
    cp_ct = pltpu.async_copy(col_hbm, ctab_v, sem_tab)
    for cp in pos_cps:
        cp.wait()

    @plsc.parallel_loop(0, BPW // 16)
    def idx_body(j):
        s = pl.ds(j * 16, 16)
        qr0 = jnp.minimum((pos_v[0, s] * VOCAB).astype(jnp.int32), VOCAB - 1)
        qc0 = jnp.minimum((pos_v[1, s] * VOCAB).astype(jnp.int32), VOCAB - 1)
        qr1 = jnp.minimum((pos_v[2, s] * VOCAB).astype(jnp.int32), VOCAB - 1)
        qc1 = jnp.minimum((pos_v[3, s] * VOCAB).astype(jnp.int32), VOCAB - 1)
        ridx_v[j // (CHUNK // 16), pl.ds((j % (CHUNK // 16)) * 16, 16)] = (
            jnp.right_shift(qr0 + qr1, 1))
        cidx_v[s] = jnp.right_shift(qc0 + qc1, 1)

    g_cps = [
        pltpu.async_copy(row_hbm.at[ridx_v.at[c]], out_v.at[c], sem_g)
        for c in range(NCHUNK)]
    cp_ct.wait()

    out_cps = []
    for c in range(NCHUNK):
        g_cps[c].wait()

        @plsc.parallel_loop(0, CHUNK // 16)
        def row_body(g):
            cvec = cidx_v[pl.ds(c * CHUNK + g * 16, 16)]

            def load_elem(e):
                ci = cvec[e]
                return [ctab_v[ci, pl.ds(k * 16, 16)] for k in range(D // 16)]

            parts = load_elem(0)
            for e in range(16):
                nxt = load_elem(e + 1) if e + 1 < 16 else None
                for k in range(D // 16):
                    plsc.addupdate(
                        out_v.at[c, g * 16 + e, pl.ds(k * 16, 16)], parts[k])
                parts = nxt

        out_cps.append(pltpu.async_copy(
            out_v.at[c], out_hbm.at[pl.ds(base + c * CHUNK, CHUNK)],
            sem_out))

    for cp in out_cps:
        cp.wait()


_mesh = plsc.VectorSubcoreMesh(core_axis_name="c", subcore_axis_name="s")

_kern = pl.kernel(
    _body,
    out_type=jax.ShapeDtypeStruct((B, D), jnp.float32),
    mesh=_mesh,
    scratch_types=[
        pltpu.VMEM((4, BPW), jnp.float32),
        pltpu.VMEM((VOCAB, D), jnp.float32),
        pltpu.VMEM((NCHUNK, CHUNK), jnp.int32),
        pltpu.VMEM((BPW,), jnp.int32),
        pltpu.VMEM((NCHUNK, CHUNK, D), jnp.float32),
        pltpu.SemaphoreType.DMA,
        pltpu.SemaphoreType.DMA,
        pltpu.SemaphoreType.DMA,
        pltpu.SemaphoreType.DMA,
    ],
)


def kernel(patch_positions, row_embedding, column_embedding):
    pos_planes = patch_positions.reshape(B, 4).T
    return _kern(pos_planes, row_embedding, column_embedding)


# speedup vs baseline: 1.3249x; 1.3249x over previous
import jax
import jax.numpy as jnp
from jax import lax
from jax.experimental import pallas as pl
from jax.experimental.pallas import tpu as pltpu
from jax.experimental.pallas import tpu_sc as plsc

VOCAB = 128
D = 128
B = 16384
NC = 2
NS = 16
NW = NC * NS
BPW = B // NW
CHUNK = 128
NCHUNK = BPW // CHUNK


def _body(pos_hbm, row_hbm, col_hbm, out_hbm,
          pos_v, rtab_v, ctab_v, ridx_v, cidx_v, out_v,
          sem_pos, sem_tab, sem_out):
    wid = lax.axis_index("s") * NC + lax.axis_index("c")
    base = wid * BPW

    pos_cps = [
        pltpu.async_copy(pos_hbm.at[a, pl.ds(base, BPW)], pos_v.at[a],
                         sem_pos)
        for a in range(4)]
    cp_rt = pltpu.async_copy(row_hbm, rtab_v, sem_tab)
    cp_ct = pltpu.async_copy(col_hbm, ctab_v, sem_tab)
    for cp in pos_cps:
        cp.wait()

    @plsc.parallel_loop(0, BPW // 16)
    def idx_body(j):
        s = pl.ds(j * 16, 16)
        qr0 = jnp.minimum((pos_v[0, s] * VOCAB).astype(jnp.int32), VOCAB - 1)
        qc0 = jnp.minimum((pos_v[1, s] * VOCAB).astype(jnp.int32), VOCAB - 1)
        qr1 = jnp.minimum((pos_v[2, s] * VOCAB).astype(jnp.int32), VOCAB - 1)
        qc1 = jnp.minimum((pos_v[3, s] * VOCAB).astype(jnp.int32), VOCAB - 1)
        ridx_v[s] = jnp.right_shift(qr0 + qr1, 1)
        cidx_v[s] = jnp.right_shift(qc0 + qc1, 1)

    cp_rt.wait()
    cp_ct.wait()

    out_cps = [None, None]
    for c in range(NCHUNK):
        buf = c % 2
        if out_cps[buf] is not None:
            out_cps[buf].wait()

        @plsc.parallel_loop(0, CHUNK // 16)
        def row_body(g):
            rvec = ridx_v[pl.ds(c * CHUNK + g * 16, 16)]
            cvec = cidx_v[pl.ds(c * CHUNK + g * 16, 16)]

            def load_elem(e):
                ri = rvec[e]
                ci = cvec[e]
                return ([rtab_v[ri, pl.ds(k * 16, 16)] for k in range(D // 16)]
                        + [ctab_v[ci, pl.ds(k * 16, 16)] for k in range(D // 16)])

            parts = load_elem(0)
            for e in range(16):
                nxt = load_elem(e + 1) if e + 1 < 16 else None
                for k in range(D // 16):
                    out_v[buf, g * 16 + e, pl.ds(k * 16, 16)] = (
                        parts[k] + parts[k + D // 16])
                parts = nxt

        out_cps[buf] = pltpu.async_copy(
            out_v.at[buf], out_hbm.at[pl.ds(base + c * CHUNK, CHUNK)],
            sem_out)

    for cp in out_cps:
        if cp is not None:
            cp.wait()


_mesh = plsc.VectorSubcoreMesh(core_axis_name="c", subcore_axis_name="s")

_kern = pl.kernel(
    _body,
    out_type=jax.ShapeDtypeStruct((B, D), jnp.float32),
    mesh=_mesh,
    scratch_types=[
        pltpu.VMEM((4, BPW), jnp.float32),
        pltpu.VMEM((VOCAB, D), jnp.float32),
        pltpu.VMEM((VOCAB, D), jnp.float32),
        pltpu.VMEM((BPW,), jnp.int32),
        pltpu.VMEM((BPW,), jnp.int32),
        pltpu.VMEM((2, CHUNK, D), jnp.float32),
        pltpu.SemaphoreType.DMA,
        pltpu.SemaphoreType.DMA,
        pltpu.SemaphoreType.DMA,
    ],
)


def kernel(patch_positions, row_embedding, column_embedding):
    pos_planes = patch_positions.reshape(B, 4).T
    return _kern(pos_planes, row_embedding, column_embedding)


# speedup vs baseline: 1.4123x; 1.0660x over previous
import jax
import jax.numpy as jnp
from jax import lax
from jax.experimental import pallas as pl
from jax.experimental.pallas import tpu as pltpu
from jax.experimental.pallas import tpu_sc as plsc

VOCAB = 128
D = 128
B = 16384
NC = 2
NS = 16
NW = NC * NS
BPW = B // NW
CHUNK = 256
NCHUNK = BPW // CHUNK


def _body(pos_hbm, row_hbm, col_hbm, out_hbm,
          pos_v, rtab_v, ctab_v, ridx_v, cidx_v, out_v,
          sem_pos, sem_tab, sem_out):
    wid = lax.axis_index("s") * NC + lax.axis_index("c")
    base = wid * BPW

    pos_cps = [
        pltpu.async_copy(pos_hbm.at[a, pl.ds(base, BPW)], pos_v.at[a],
                         sem_pos)
        for a in range(4)]
    cp_rt = pltpu.async_copy(row_hbm, rtab_v, sem_tab)
    cp_ct = pltpu.async_copy(col_hbm, ctab_v, sem_tab)
    for cp in pos_cps:
        cp.wait()

    @plsc.parallel_loop(0, BPW // 16)
    def idx_body(j):
        s = pl.ds(j * 16, 16)
        qr0 = jnp.minimum((pos_v[0, s] * VOCAB).astype(jnp.int32), VOCAB - 1)
        qc0 = jnp.minimum((pos_v[1, s] * VOCAB).astype(jnp.int32), VOCAB - 1)
        qr1 = jnp.minimum((pos_v[2, s] * VOCAB).astype(jnp.int32), VOCAB - 1)
        qc1 = jnp.minimum((pos_v[3, s] * VOCAB).astype(jnp.int32), VOCAB - 1)
        ridx_v[s] = jnp.right_shift(qr0 + qr1, 1)
        cidx_v[s] = jnp.right_shift(qc0 + qc1, 1)

    cp_rt.wait()
    cp_ct.wait()

    out_cps = [None, None]
    for c in range(NCHUNK):
        buf = c % 2
        if out_cps[buf] is not None:
            out_cps[buf].wait()

        @plsc.parallel_loop(0, CHUNK // 16)
        def row_body(g):
            rvec = ridx_v[pl.ds(c * CHUNK + g * 16, 16)]
            cvec = cidx_v[pl.ds(c * CHUNK + g * 16, 16)]

            def load_elem(e):
                ri = rvec[e]
                ci = cvec[e]
                return ([rtab_v[ri, pl.ds(k * 16, 16)] for k in range(D // 16)]
                        + [ctab_v[ci, pl.ds(k * 16, 16)] for k in range(D // 16)])

            parts = load_elem(0)
            for e in range(16):
                nxt = load_elem(e + 1) if e + 1 < 16 else None
                for k in range(D // 16):
                    out_v[buf, g * 16 + e, pl.ds(k * 16, 16)] = (
                        parts[k] + parts[k + D // 16])
                parts = nxt

        out_cps[buf] = pltpu.async_copy(
            out_v.at[buf], out_hbm.at[pl.ds(base + c * CHUNK, CHUNK)],
            sem_out)

    for cp in out_cps:
        if cp is not None:
            cp.wait()


_mesh = plsc.VectorSubcoreMesh(core_axis_name="c", subcore_axis_name="s")

_kern = pl.kernel(
    _body,
    out_type=jax.ShapeDtypeStruct((B, D), jnp.float32),
    mesh=_mesh,
    scratch_types=[
        pltpu.VMEM((4, BPW), jnp.float32),
        pltpu.VMEM((VOCAB, D), jnp.float32),
        pltpu.VMEM((VOCAB, D), jnp.float32),
        pltpu.VMEM((BPW,), jnp.int32),
        pltpu.VMEM((BPW,), jnp.int32),
        pltpu.VMEM((2, CHUNK, D), jnp.float32),
        pltpu.SemaphoreType.DMA,
        pltpu.SemaphoreType.DMA,
        pltpu.SemaphoreType.DMA,
    ],
)


def kernel(patch_positions, row_embedding, column_embedding):
    pos_planes = patch_positions.reshape(B, 4).T
    return _kern(pos_planes, row_embedding, column_embedding)
